# baseline (speedup 1.0000x reference)
import functools

import jax
import jax.numpy as jnp
from jax import lax
from jax.experimental import pallas as pl
from jax.experimental.pallas import tpu as pltpu
from jax.experimental.pallas import tpu_sc as plsc

_N, _E, _D, _H, _C = 10000, 320000, 128, 128, 40
_NC, _NS = 2, 16
_NW = _NC * _NS
_EPW = _E // _NW
_CHUNK = 128
_EPWP = 10240
_PAD = _EPWP - _EPW
_NCHUNK = _EPWP // _CHUNK
_NBLK = 5
_BCH = _NCHUNK // _NBLK
_NBUF = 2
_NACC = _N + _NS
_RPS = 624
_TAIL0 = _NS * _RPS
_TAIL = _N - _TAIL0


def _agg_body(h_hbm, src_hbm, dst_hbm, zero_hbm, out_hbm, *refs):
    sidx = refs[0:2]
    didx = refs[2:4]
    rows = refs[4:4 + _NBUF]
    acc = refs[4 + _NBUF]
    is0, is1 = refs[5 + _NBUF:7 + _NBUF]
    gs = refs[7 + _NBUF:7 + 2 * _NBUF]
    ss = refs[7 + 2 * _NBUF:7 + 3 * _NBUF]

    c = lax.axis_index("c")
    s = lax.axis_index("s")
    wid = c * _NS + s
    row0 = s * _RPS
    pltpu.sync_copy(zero_hbm.at[pl.ds(row0, _RPS)], acc.at[pl.ds(row0, _RPS)])

    @pl.when(s == _NS - 1)
    def _zero_tail():
        pltpu.sync_copy(zero_hbm.at[pl.ds(_TAIL0, _TAIL)],
                        acc.at[pl.ds(_TAIL0, _TAIL)])

    pltpu.sync_copy(src_hbm.at[wid, pl.ds(0, _BCH)], sidx[0])
    pltpu.sync_copy(dst_hbm.at[wid, pl.ds(0, _BCH)], didx[0])
    plsc.subcore_barrier()

    def g_start(si, j, k):
        pltpu.async_copy(h_hbm.at[si.at[j]], rows[k], gs[k])

    def g_wait(si, j, k):
        pltpu.make_async_copy(h_hbm.at[si.at[j]], rows[k], gs[k]).wait()

    def s_start(di, j, k):
        pltpu.async_copy(rows[k], acc.at[di.at[j]], ss[k], add=True)

    def s_wait(di, j, k):
        pltpu.make_async_copy(rows[k], acc.at[di.at[j]], ss[k]).wait()

    for b in range(_NBLK):
        si = sidx[b % 2]
        di = didx[b % 2]
        if b + 1 < _NBLK:
            pltpu.async_copy(src_hbm.at[wid, pl.ds((b + 1) * _BCH, _BCH)],
                             sidx[(b + 1) % 2], is0)
            pltpu.async_copy(dst_hbm.at[wid, pl.ds((b + 1) * _BCH, _BCH)],
                             didx[(b + 1) % 2], is1)
        for k in range(_NBUF):
            g_start(si, k, k)

        def quad(q, carry, si=si, di=di):
            j = _NBUF * q
            for k in range(_NBUF):
                g_wait(si, j + k, k)
                s_start(di, j + k, k)
            for k in range(_NBUF):
                s_wait(di, j + k, k)
                g_start(si, j + _NBUF + k, k)
            return carry

        lax.fori_loop(0, _BCH // _NBUF - 1, quad, 0)
        jlast = _BCH - _NBUF
        for k in range(_NBUF):
            g_wait(si, jlast + k, k)
            s_start(di, jlast + k, k)
        for k in range(_NBUF):
            s_wait(di, jlast + k, k)
        if b + 1 < _NBLK:
            pltpu.make_async_copy(
                src_hbm.at[wid, pl.ds((b + 1) * _BCH, _BCH)],
                sidx[(b + 1) % 2], is0).wait()
            pltpu.make_async_copy(
                dst_hbm.at[wid, pl.ds((b + 1) * _BCH, _BCH)],
                didx[(b + 1) % 2], is1).wait()

    plsc.subcore_barrier()
    pltpu.sync_copy(acc.at[pl.ds(row0, _RPS)],
                    out_hbm.at[pl.ds(c * _N + row0, _RPS)])

    @pl.when(s == _NS - 1)
    def _write_tail():
        pltpu.sync_copy(acc.at[pl.ds(_TAIL0, _TAIL)],
                        out_hbm.at[pl.ds(c * _N + _TAIL0, _TAIL)])


_agg = functools.partial(
    pl.kernel,
    mesh=plsc.VectorSubcoreMesh(core_axis_name="c", subcore_axis_name="s"),
    out_type=jax.ShapeDtypeStruct((_NC * _N, _D), jnp.float32),
    scratch_types=(
        [pltpu.VMEM((_BCH, _CHUNK), jnp.int32)] * 4
        + [pltpu.VMEM((_CHUNK, _D), jnp.float32)] * _NBUF
        + [pltpu.VMEM_SHARED((_NACC, _D), jnp.float32)]
        + [pltpu.SemaphoreType.DMA] * (2 + 2 * _NBUF)
    ),
)(_agg_body)


_DOT = functools.partial(jnp.dot, preferred_element_type=jnp.float32,
                         precision=lax.Precision.HIGHEST)


def _mlp(h, wa_ref, ba_ref, wb_ref, bb_ref):
    h = jnp.maximum(_DOT(h, wa_ref[...]) + ba_ref[...], 0.0)
    return jnp.maximum(_DOT(h, wb_ref[...]) + bb_ref[...], 0.0)


def _dense_body(x_ref, a_ref, wa_ref, ba_ref, wb_ref, bb_ref,
                g_ref, be_ref, out_ref):
    h = x_ref[...] + a_ref[:_N, :] + a_ref[_N:, :]
    h = _mlp(h, wa_ref, ba_ref, wb_ref, bb_ref)
    mu = jnp.mean(h, axis=0, keepdims=True)
    var = jnp.mean((h - mu) * (h - mu), axis=0, keepdims=True)
    h = g_ref[...] * (h - mu) / jnp.sqrt(var + 1e-5) + be_ref[...]
    out_ref[...] = jnp.maximum(h, 0.0)


def _final_body(x_ref, a_ref, wa_ref, ba_ref, wb_ref, bb_ref,
                wl_ref, bl_ref, out_ref):
    h = x_ref[...] + a_ref[:_N, :] + a_ref[_N:, :]
    h = _mlp(h, wa_ref, ba_ref, wb_ref, bb_ref)
    logits = _DOT(h, wl_ref[...]) + bl_ref[...]
    m = jnp.max(logits, axis=-1, keepdims=True)
    z = logits - m
    out_ref[...] = z - jnp.log(jnp.sum(jnp.exp(z), axis=-1, keepdims=True))


_dense = pl.pallas_call(
    _dense_body, out_shape=jax.ShapeDtypeStruct((_N, _H), jnp.float32))
_final = pl.pallas_call(
    _final_body, out_shape=jax.ShapeDtypeStruct((_N, _C), jnp.float32))


def kernel(x, edge_index, W0a, b0a, W0b, b0b, W1a, b1a, W1b, b1b,
           W2a, b2a, W2b, b2b, g0, be0, g1, be1, Wlin, blin):
    src = edge_index[0].astype(jnp.int32).reshape(_NW, _EPW)
    dst = edge_index[1].astype(jnp.int32).reshape(_NW, _EPW)
    pad_src = (jnp.arange(_NW * _PAD, dtype=jnp.int32) % _N).reshape(_NW, _PAD)
    pad_dst = jnp.broadcast_to(
        _N + (jnp.arange(_NW, dtype=jnp.int32) % _NS)[:, None], (_NW, _PAD))
    src = jnp.concatenate([src, pad_src], 1).reshape(_NW, _NCHUNK, _CHUNK)
    dst = jnp.concatenate([dst, pad_dst], 1).reshape(_NW, _NCHUNK, _CHUNK)
    zeros = jnp.zeros((_N, _D), jnp.float32)
    r1 = lambda v: v.reshape(1, -1)

    a0 = _agg(x, src, dst, zeros)
    h0 = _dense(x, a0, W0a, r1(b0a), W0b, r1(b0b), r1(g0), r1(be0))
    a1 = _agg(h0, src, dst, zeros)
    h1 = _dense(h0, a1, W1a, r1(b1a), W1b, r1(b1b), r1(g1), r1(be1))
    a2 = _agg(h1, src, dst, zeros)
    return _final(h1, a2, W2a, r1(b2a), W2b, r1(b2b), Wlin, r1(blin))


# speedup vs baseline: 1.1992x; 1.1992x over previous
import functools

import jax
import jax.numpy as jnp
from jax import lax
from jax.experimental import pallas as pl
from jax.experimental.pallas import tpu as pltpu
from jax.experimental.pallas import tpu_sc as plsc

_N, _E, _D, _H, _C = 10000, 320000, 128, 128, 40
_NC, _NS = 2, 16
_NW = _NC * _NS
_EPW = _E // _NW
_CHUNK = 64
_EPWP = 10240
_PAD = _EPWP - _EPW
_NCHUNK = _EPWP // _CHUNK
_NBLK = 5
_BCH = _NCHUNK // _NBLK
_NBUF = 4
_NACC = _N + _NS
_RPS = 624
_TAIL0 = _NS * _RPS
_TAIL = _N - _TAIL0


def _agg_body(h_hbm, src_hbm, dst_hbm, out_hbm, *refs):
    sidx = refs[0:2]
    didx = refs[2:4]
    rows = refs[4:4 + _NBUF]
    acc = refs[4 + _NBUF]
    is0, is1 = refs[5 + _NBUF:7 + _NBUF]
    gs = refs[7 + _NBUF:7 + 2 * _NBUF]
    ss = refs[7 + 2 * _NBUF:7 + 3 * _NBUF]

    c = lax.axis_index("c")
    s = lax.axis_index("s")
    wid = c * _NS + s
    row0 = s * _RPS

    z16 = jnp.zeros((16,), jnp.float32)

    def zfill(r, carry):
        for cc in range(_D // 16):
            rows[0][r, pl.ds(cc * 16, 16)] = z16
        return carry

    lax.fori_loop(0, _CHUNK, zfill, 0)
    for q in range(_RPS // _CHUNK):
        pltpu.sync_copy(rows[0], acc.at[pl.ds(row0 + q * _CHUNK, _CHUNK)])
    _REM = _RPS % _CHUNK
    if _REM:
        pltpu.sync_copy(rows[0].at[pl.ds(0, _REM)],
                        acc.at[pl.ds(row0 + _RPS - _REM, _REM)])

    @pl.when(s == _NS - 1)
    def _zero_tail():
        pltpu.sync_copy(rows[0].at[pl.ds(0, _TAIL)],
                        acc.at[pl.ds(_TAIL0, _TAIL)])

    pltpu.sync_copy(src_hbm.at[wid, pl.ds(0, _BCH)], sidx[0])
    pltpu.sync_copy(dst_hbm.at[wid, pl.ds(0, _BCH)], didx[0])

    def g_start(si, j, k):
        pltpu.async_copy(h_hbm.at[si.at[j]], rows[k], gs[k])

    def g_wait(si, j, k):
        pltpu.make_async_copy(h_hbm.at[si.at[j]], rows[k], gs[k]).wait()

    def s_start(di, j, k):
        pltpu.async_copy(rows[k], acc.at[di.at[j]], ss[k], add=True)

    def s_wait(di, j, k):
        pltpu.make_async_copy(rows[k], acc.at[di.at[j]], ss[k]).wait()

    for k in range(_NBUF):
        g_start(sidx[0], k, k)
    plsc.subcore_barrier()

    for b in range(_NBLK):
        si = sidx[b % 2]
        di = didx[b % 2]
        if b + 1 < _NBLK:
            pltpu.async_copy(src_hbm.at[wid, pl.ds((b + 1) * _BCH, _BCH)],
                             sidx[(b + 1) % 2], is0)
            pltpu.async_copy(dst_hbm.at[wid, pl.ds((b + 1) * _BCH, _BCH)],
                             didx[(b + 1) % 2], is1)
        if b > 0:
            for k in range(_NBUF):
                g_start(si, k, k)

        def quad(q, carry, si=si, di=di):
            j = _NBUF * q
            for k in range(_NBUF):
                g_wait(si, j + k, k)
                s_start(di, j + k, k)
            for k in range(_NBUF):
                s_wait(di, j + k, k)
                g_start(si, j + _NBUF + k, k)
            return carry

        lax.fori_loop(0, _BCH // _NBUF - 1, quad, 0)
        jlast = _BCH - _NBUF
        for k in range(_NBUF):
            g_wait(si, jlast + k, k)
            s_start(di, jlast + k, k)
        for k in range(_NBUF):
            s_wait(di, jlast + k, k)
        if b + 1 < _NBLK:
            pltpu.make_async_copy(
                src_hbm.at[wid, pl.ds((b + 1) * _BCH, _BCH)],
                sidx[(b + 1) % 2], is0).wait()
            pltpu.make_async_copy(
                dst_hbm.at[wid, pl.ds((b + 1) * _BCH, _BCH)],
                didx[(b + 1) % 2], is1).wait()

    plsc.subcore_barrier()
    pltpu.sync_copy(acc.at[pl.ds(row0, _RPS)],
                    out_hbm.at[pl.ds(c * _N + row0, _RPS)])

    @pl.when(s == _NS - 1)
    def _write_tail():
        pltpu.sync_copy(acc.at[pl.ds(_TAIL0, _TAIL)],
                        out_hbm.at[pl.ds(c * _N + _TAIL0, _TAIL)])


_agg = functools.partial(
    pl.kernel,
    mesh=plsc.VectorSubcoreMesh(core_axis_name="c", subcore_axis_name="s"),
    out_type=jax.ShapeDtypeStruct((_NC * _N, _D), jnp.float32),
    scratch_types=(
        [pltpu.VMEM((_BCH, _CHUNK), jnp.int32)] * 4
        + [pltpu.VMEM((_CHUNK, _D), jnp.float32)] * _NBUF
        + [pltpu.VMEM_SHARED((_NACC, _D), jnp.float32)]
        + [pltpu.SemaphoreType.DMA] * (2 + 2 * _NBUF)
    ),
)(_agg_body)


_DOT = functools.partial(jnp.dot, preferred_element_type=jnp.float32,
                         precision=lax.Precision.HIGHEST)


def _mlp(h, wa_ref, ba_ref, wb_ref, bb_ref):
    h = jnp.maximum(_DOT(h, wa_ref[...]) + ba_ref[...], 0.0)
    return jnp.maximum(_DOT(h, wb_ref[...]) + bb_ref[...], 0.0)


def _dense_body(x_ref, a_ref, wa_ref, ba_ref, wb_ref, bb_ref,
                g_ref, be_ref, out_ref):
    h = x_ref[...] + a_ref[:_N, :] + a_ref[_N:, :]
    h = _mlp(h, wa_ref, ba_ref, wb_ref, bb_ref)
    mu = jnp.mean(h, axis=0, keepdims=True)
    var = jnp.mean((h - mu) * (h - mu), axis=0, keepdims=True)
    h = g_ref[...] * (h - mu) / jnp.sqrt(var + 1e-5) + be_ref[...]
    out_ref[...] = jnp.maximum(h, 0.0)


def _final_body(x_ref, a_ref, wa_ref, ba_ref, wb_ref, bb_ref,
                wl_ref, bl_ref, out_ref):
    h = x_ref[...] + a_ref[:_N, :] + a_ref[_N:, :]
    h = _mlp(h, wa_ref, ba_ref, wb_ref, bb_ref)
    logits = _DOT(h, wl_ref[...]) + bl_ref[...]
    m = jnp.max(logits, axis=-1, keepdims=True)
    z = logits - m
    out_ref[...] = z - jnp.log(jnp.sum(jnp.exp(z), axis=-1, keepdims=True))


_dense = pl.pallas_call(
    _dense_body, out_shape=jax.ShapeDtypeStruct((_N, _H), jnp.float32))
_final = pl.pallas_call(
    _final_body, out_shape=jax.ShapeDtypeStruct((_N, _C), jnp.float32))


def kernel(x, edge_index, W0a, b0a, W0b, b0b, W1a, b1a, W1b, b1b,
           W2a, b2a, W2b, b2b, g0, be0, g1, be1, Wlin, blin):
    src = edge_index[0].astype(jnp.int32).reshape(_NW, _EPW)
    dst = edge_index[1].astype(jnp.int32).reshape(_NW, _EPW)
    pad_src = (jnp.arange(_NW * _PAD, dtype=jnp.int32) % _N).reshape(_NW, _PAD)
    pad_dst = jnp.broadcast_to(
        _N + (jnp.arange(_NW, dtype=jnp.int32) % _NS)[:, None], (_NW, _PAD))
    src = jnp.concatenate([src, pad_src], 1).reshape(_NW, _NCHUNK, _CHUNK)
    dst = jnp.concatenate([dst, pad_dst], 1).reshape(_NW, _NCHUNK, _CHUNK)
    r1 = lambda v: v.reshape(1, -1)

    a0 = _agg(x, src, dst)
    h0 = _dense(x, a0, W0a, r1(b0a), W0b, r1(b0b), r1(g0), r1(be0))
    a1 = _agg(h0, src, dst)
    h1 = _dense(h0, a1, W1a, r1(b1a), W1b, r1(b1b), r1(g1), r1(be1))
    a2 = _agg(h1, src, dst)
    return _final(h1, a2, W2a, r1(b2a), W2b, r1(b2b), Wlin, r1(blin))


# speedup vs baseline: 1.3582x; 1.1326x over previous
import functools

import jax
import jax.numpy as jnp
from jax import lax
from jax.experimental import pallas as pl
from jax.experimental.pallas import tpu as pltpu
from jax.experimental.pallas import tpu_sc as plsc

_N, _E, _D, _H, _C = 10000, 320000, 128, 128, 40
_NC, _NS = 2, 16
_NW = _NC * _NS
_EPW = _E // _NW
_CHUNK = 64
_EPWP = 10240
_PAD = _EPWP - _EPW
_NCHUNK = _EPWP // _CHUNK
_NBLK = 5
_BCH = _NCHUNK // _NBLK
_NBUF = 4
_NACC = _N + _NS
_RPS = 624
_TAIL0 = _NS * _RPS
_TAIL = _N - _TAIL0


def _agg_body(h_hbm, src_hbm, dst_hbm, out_hbm, *refs):
    sidx = refs[0:2]
    didx = refs[2:4]
    rows = refs[4:4 + _NBUF]
    acc = refs[4 + _NBUF]
    is0, is1 = refs[5 + _NBUF:7 + _NBUF]
    gs = refs[7 + _NBUF:7 + 2 * _NBUF]
    ss = refs[7 + 2 * _NBUF:7 + 3 * _NBUF]

    c = lax.axis_index("c")
    s = lax.axis_index("s")
    wid = c * _NS + s
    row0 = s * _RPS

    z16 = jnp.zeros((16,), jnp.float32)

    def zfill(r, carry):
        for cc in range(_D // 16):
            rows[0][r, pl.ds(cc * 16, 16)] = z16
        return carry

    lax.fori_loop(0, _CHUNK, zfill, 0)
    for q in range(_RPS // _CHUNK):
        pltpu.sync_copy(rows[0], acc.at[pl.ds(row0 + q * _CHUNK, _CHUNK)])
    _REM = _RPS % _CHUNK
    if _REM:
        pltpu.sync_copy(rows[0].at[pl.ds(0, _REM)],
                        acc.at[pl.ds(row0 + _RPS - _REM, _REM)])

    @pl.when(s == _NS - 1)
    def _zero_tail():
        pltpu.sync_copy(rows[0].at[pl.ds(0, _TAIL)],
                        acc.at[pl.ds(_TAIL0, _TAIL)])

    pltpu.sync_copy(src_hbm.at[wid, pl.ds(0, _BCH)], sidx[0])
    pltpu.sync_copy(dst_hbm.at[wid, pl.ds(0, _BCH)], didx[0])

    def g_start(si, j, k):
        pltpu.async_copy(h_hbm.at[si.at[j]], rows[k], gs[k])

    def g_wait(si, j, k):
        pltpu.make_async_copy(h_hbm.at[si.at[j]], rows[k], gs[k]).wait()

    def s_start(di, j, k):
        pltpu.async_copy(rows[k], acc.at[di.at[j]], ss[k], add=True)

    def s_wait(di, j, k):
        pltpu.make_async_copy(rows[k], acc.at[di.at[j]], ss[k]).wait()

    for k in range(_NBUF):
        g_start(sidx[0], k, k)
    plsc.subcore_barrier()

    for b in range(_NBLK):
        si = sidx[b % 2]
        di = didx[b % 2]
        if b + 1 < _NBLK:
            pltpu.async_copy(src_hbm.at[wid, pl.ds((b + 1) * _BCH, _BCH)],
                             sidx[(b + 1) % 2], is0)
            pltpu.async_copy(dst_hbm.at[wid, pl.ds((b + 1) * _BCH, _BCH)],
                             didx[(b + 1) % 2], is1)
        if b > 0:
            for k in range(_NBUF):
                g_start(si, k, k)

        def quad(q, carry, si=si, di=di):
            j = _NBUF * q
            for k in range(_NBUF):
                g_wait(si, j + k, k)
                s_start(di, j + k, k)
            for k in range(_NBUF):
                s_wait(di, j + k, k)
                g_start(si, j + _NBUF + k, k)
            return carry

        lax.fori_loop(0, _BCH // _NBUF - 1, quad, 0)
        jlast = _BCH - _NBUF
        for k in range(_NBUF):
            g_wait(si, jlast + k, k)
            s_start(di, jlast + k, k)
        for k in range(_NBUF):
            s_wait(di, jlast + k, k)
        if b + 1 < _NBLK:
            pltpu.make_async_copy(
                src_hbm.at[wid, pl.ds((b + 1) * _BCH, _BCH)],
                sidx[(b + 1) % 2], is0).wait()
            pltpu.make_async_copy(
                dst_hbm.at[wid, pl.ds((b + 1) * _BCH, _BCH)],
                didx[(b + 1) % 2], is1).wait()

    plsc.subcore_barrier()
    pltpu.sync_copy(acc.at[pl.ds(row0, _RPS)],
                    out_hbm.at[pl.ds(c * _N + row0, _RPS)])

    @pl.when(s == _NS - 1)
    def _write_tail():
        pltpu.sync_copy(acc.at[pl.ds(_TAIL0, _TAIL)],
                        out_hbm.at[pl.ds(c * _N + _TAIL0, _TAIL)])


_agg = functools.partial(
    pl.kernel,
    mesh=plsc.VectorSubcoreMesh(core_axis_name="c", subcore_axis_name="s"),
    out_type=jax.ShapeDtypeStruct((_NC * _N, _D), jnp.float32),
    scratch_types=(
        [pltpu.VMEM((_BCH, _CHUNK), jnp.int32)] * 4
        + [pltpu.VMEM((_CHUNK, _D), jnp.float32)] * _NBUF
        + [pltpu.VMEM_SHARED((_NACC, _D), jnp.float32)]
        + [pltpu.SemaphoreType.DMA] * (2 + 2 * _NBUF)
    ),
)(_agg_body)


_DOT = functools.partial(jnp.dot, preferred_element_type=jnp.float32)


def _mlp(h, wa_ref, ba_ref, wb_ref, bb_ref):
    h = jnp.maximum(_DOT(h, wa_ref[...]) + ba_ref[...], 0.0)
    return jnp.maximum(_DOT(h, wb_ref[...]) + bb_ref[...], 0.0)


def _dense_body(x_ref, a_ref, wa_ref, ba_ref, wb_ref, bb_ref,
                g_ref, be_ref, out_ref):
    h = x_ref[...] + a_ref[:_N, :] + a_ref[_N:, :]
    h = _mlp(h, wa_ref, ba_ref, wb_ref, bb_ref)
    mu = jnp.mean(h, axis=0, keepdims=True)
    var = jnp.mean((h - mu) * (h - mu), axis=0, keepdims=True)
    h = g_ref[...] * (h - mu) / jnp.sqrt(var + 1e-5) + be_ref[...]
    out_ref[...] = jnp.maximum(h, 0.0)


def _final_body(x_ref, a_ref, wa_ref, ba_ref, wb_ref, bb_ref,
                wl_ref, bl_ref, out_ref):
    h = x_ref[...] + a_ref[:_N, :] + a_ref[_N:, :]
    h = _mlp(h, wa_ref, ba_ref, wb_ref, bb_ref)
    logits = _DOT(h, wl_ref[...]) + bl_ref[...]
    m = jnp.max(logits, axis=-1, keepdims=True)
    z = logits - m
    out_ref[...] = z - jnp.log(jnp.sum(jnp.exp(z), axis=-1, keepdims=True))


_dense = pl.pallas_call(
    _dense_body, out_shape=jax.ShapeDtypeStruct((_N, _H), jnp.float32))
_final = pl.pallas_call(
    _final_body, out_shape=jax.ShapeDtypeStruct((_N, _C), jnp.float32))


def kernel(x, edge_index, W0a, b0a, W0b, b0b, W1a, b1a, W1b, b1b,
           W2a, b2a, W2b, b2b, g0, be0, g1, be1, Wlin, blin):
    src = edge_index[0].astype(jnp.int32).reshape(_NW, _EPW)
    dst = edge_index[1].astype(jnp.int32).reshape(_NW, _EPW)
    pad_src = (jnp.arange(_NW * _PAD, dtype=jnp.int32) % _N).reshape(_NW, _PAD)
    pad_dst = jnp.broadcast_to(
        _N + (jnp.arange(_NW, dtype=jnp.int32) % _NS)[:, None], (_NW, _PAD))
    src = jnp.concatenate([src, pad_src], 1).reshape(_NW, _NCHUNK, _CHUNK)
    dst = jnp.concatenate([dst, pad_dst], 1).reshape(_NW, _NCHUNK, _CHUNK)
    r1 = lambda v: v.reshape(1, -1)

    a0 = _agg(x, src, dst)
    h0 = _dense(x, a0, W0a, r1(b0a), W0b, r1(b0b), r1(g0), r1(be0))
    a1 = _agg(h0, src, dst)
    h1 = _dense(h0, a1, W1a, r1(b1a), W1b, r1(b1b), r1(g1), r1(be1))
    a2 = _agg(h1, src, dst)
    return _final(h1, a2, W2a, r1(b2a), W2b, r1(b2b), Wlin, r1(blin))


# speedup vs baseline: 1.3809x; 1.0168x over previous
import functools

import jax
import jax.numpy as jnp
from jax import lax
from jax.experimental import pallas as pl
from jax.experimental.pallas import tpu as pltpu
from jax.experimental.pallas import tpu_sc as plsc

_N, _E, _D, _H, _C = 10000, 320000, 128, 128, 40
_NC, _NS = 2, 16
_NW = _NC * _NS
_EPW = _E // _NW
_CHUNK = 64
_EPWP = 10240
_PAD = _EPWP - _EPW
_NCHUNK = _EPWP // _CHUNK
_NBLK = 5
_BCH = _NCHUNK // _NBLK
_NBUF = 4
_NACC = _N + _NS
_RPS = 624
_TAIL0 = _NS * _RPS
_TAIL = _N - _TAIL0


def _agg_body(h_hbm, src_hbm, dst_hbm, out_hbm, *refs):
    sidx = refs[0:2]
    didx = refs[2:4]
    rows = refs[4:4 + _NBUF]
    acc = refs[4 + _NBUF]
    is0, is1 = refs[5 + _NBUF:7 + _NBUF]
    gs = refs[7 + _NBUF:7 + 2 * _NBUF]
    ss = refs[7 + 2 * _NBUF:7 + 3 * _NBUF]

    c = lax.axis_index("c")
    s = lax.axis_index("s")
    wid = c * _NS + s
    row0 = s * _RPS

    z16 = jnp.zeros((16,), jnp.float32)

    def zfill(r, carry):
        for cc in range(_D // 16):
            rows[0][r, pl.ds(cc * 16, 16)] = z16
        return carry

    lax.fori_loop(0, _CHUNK, zfill, 0)
    for q in range(_RPS // _CHUNK):
        pltpu.sync_copy(rows[0], acc.at[pl.ds(row0 + q * _CHUNK, _CHUNK)])
    _REM = _RPS % _CHUNK
    if _REM:
        pltpu.sync_copy(rows[0].at[pl.ds(0, _REM)],
                        acc.at[pl.ds(row0 + _RPS - _REM, _REM)])

    @pl.when(s == _NS - 1)
    def _zero_tail():
        pltpu.sync_copy(rows[0].at[pl.ds(0, _TAIL)],
                        acc.at[pl.ds(_TAIL0, _TAIL)])

    pltpu.sync_copy(src_hbm.at[wid, pl.ds(0, _BCH)], sidx[0])
    pltpu.sync_copy(dst_hbm.at[wid, pl.ds(0, _BCH)], didx[0])

    def g_start(si, j, k):
        pltpu.async_copy(h_hbm.at[si.at[j]], rows[k], gs[k])

    def g_wait(si, j, k):
        pltpu.make_async_copy(h_hbm.at[si.at[j]], rows[k], gs[k]).wait()

    def s_start(di, j, k):
        pltpu.async_copy(rows[k], acc.at[di.at[j]], ss[k], add=True)

    def s_wait(di, j, k):
        pltpu.make_async_copy(rows[k], acc.at[di.at[j]], ss[k]).wait()

    for k in range(_NBUF):
        g_start(sidx[0], k, k)
    plsc.subcore_barrier()

    for b in range(_NBLK):
        si = sidx[b % 2]
        di = didx[b % 2]
        if b + 1 < _NBLK:
            pltpu.async_copy(src_hbm.at[wid, pl.ds((b + 1) * _BCH, _BCH)],
                             sidx[(b + 1) % 2], is0)
            pltpu.async_copy(dst_hbm.at[wid, pl.ds((b + 1) * _BCH, _BCH)],
                             didx[(b + 1) % 2], is1)
        def quad(q, carry, si=si, di=di):
            j = _NBUF * q
            for k in range(_NBUF):
                g_wait(si, j + k, k)
                s_start(di, j + k, k)
            for k in range(_NBUF):
                s_wait(di, j + k, k)
                g_start(si, j + _NBUF + k, k)
            return carry

        lax.fori_loop(0, _BCH // _NBUF - 1, quad, 0)
        jlast = _BCH - _NBUF
        for k in range(_NBUF):
            g_wait(si, jlast + k, k)
            s_start(di, jlast + k, k)
        if b + 1 < _NBLK:
            pltpu.make_async_copy(
                src_hbm.at[wid, pl.ds((b + 1) * _BCH, _BCH)],
                sidx[(b + 1) % 2], is0).wait()
            pltpu.make_async_copy(
                dst_hbm.at[wid, pl.ds((b + 1) * _BCH, _BCH)],
                didx[(b + 1) % 2], is1).wait()
            for k in range(_NBUF):
                s_wait(di, jlast + k, k)
                g_start(sidx[(b + 1) % 2], k, k)
        else:
            for k in range(_NBUF):
                s_wait(di, jlast + k, k)

    plsc.subcore_barrier()
    pltpu.sync_copy(acc.at[pl.ds(row0, _RPS)],
                    out_hbm.at[pl.ds(c * _N + row0, _RPS)])

    @pl.when(s == _NS - 1)
    def _write_tail():
        pltpu.sync_copy(acc.at[pl.ds(_TAIL0, _TAIL)],
                        out_hbm.at[pl.ds(c * _N + _TAIL0, _TAIL)])


_agg = functools.partial(
    pl.kernel,
    mesh=plsc.VectorSubcoreMesh(core_axis_name="c", subcore_axis_name="s"),
    out_type=jax.ShapeDtypeStruct((_NC * _N, _D), jnp.float32),
    scratch_types=(
        [pltpu.VMEM((_BCH, _CHUNK), jnp.int32)] * 4
        + [pltpu.VMEM((_CHUNK, _D), jnp.float32)] * _NBUF
        + [pltpu.VMEM_SHARED((_NACC, _D), jnp.float32)]
        + [pltpu.SemaphoreType.DMA] * (2 + 2 * _NBUF)
    ),
)(_agg_body)


_DOT = functools.partial(jnp.dot, preferred_element_type=jnp.float32)


def _mlp(h, wa_ref, ba_ref, wb_ref, bb_ref):
    h = jnp.maximum(_DOT(h, wa_ref[...]) + ba_ref[...], 0.0)
    return jnp.maximum(_DOT(h, wb_ref[...]) + bb_ref[...], 0.0)


def _dense_body(x_ref, a_ref, wa_ref, ba_ref, wb_ref, bb_ref,
                g_ref, be_ref, out_ref):
    h = x_ref[...] + a_ref[:_N, :] + a_ref[_N:, :]
    h = _mlp(h, wa_ref, ba_ref, wb_ref, bb_ref)
    mu = jnp.mean(h, axis=0, keepdims=True)
    var = jnp.mean((h - mu) * (h - mu), axis=0, keepdims=True)
    h = g_ref[...] * (h - mu) / jnp.sqrt(var + 1e-5) + be_ref[...]
    out_ref[...] = jnp.maximum(h, 0.0)


def _final_body(x_ref, a_ref, wa_ref, ba_ref, wb_ref, bb_ref,
                wl_ref, bl_ref, out_ref):
    h = x_ref[...] + a_ref[:_N, :] + a_ref[_N:, :]
    h = _mlp(h, wa_ref, ba_ref, wb_ref, bb_ref)
    logits = _DOT(h, wl_ref[...]) + bl_ref[...]
    m = jnp.max(logits, axis=-1, keepdims=True)
    z = logits - m
    out_ref[...] = z - jnp.log(jnp.sum(jnp.exp(z), axis=-1, keepdims=True))


_dense = pl.pallas_call(
    _dense_body, out_shape=jax.ShapeDtypeStruct((_N, _H), jnp.float32))
_final = pl.pallas_call(
    _final_body, out_shape=jax.ShapeDtypeStruct((_N, _C), jnp.float32))


def kernel(x, edge_index, W0a, b0a, W0b, b0b, W1a, b1a, W1b, b1b,
           W2a, b2a, W2b, b2b, g0, be0, g1, be1, Wlin, blin):
    src = edge_index[0].astype(jnp.int32).reshape(_NW, _EPW)
    dst = edge_index[1].astype(jnp.int32).reshape(_NW, _EPW)
    pad_src = (jnp.arange(_NW * _PAD, dtype=jnp.int32) % _N).reshape(_NW, _PAD)
    pad_dst = jnp.broadcast_to(
        _N + (jnp.arange(_NW, dtype=jnp.int32) % _NS)[:, None], (_NW, _PAD))
    src = jnp.concatenate([src, pad_src], 1).reshape(_NW, _NCHUNK, _CHUNK)
    dst = jnp.concatenate([dst, pad_dst], 1).reshape(_NW, _NCHUNK, _CHUNK)
    r1 = lambda v: v.reshape(1, -1)

    a0 = _agg(x, src, dst)
    h0 = _dense(x, a0, W0a, r1(b0a), W0b, r1(b0b), r1(g0), r1(be0))
    a1 = _agg(h0, src, dst)
    h1 = _dense(h0, a1, W1a, r1(b1a), W1b, r1(b1b), r1(g1), r1(be1))
    a2 = _agg(h1, src, dst)
    return _final(h1, a2, W2a, r1(b2a), W2b, r1(b2b), Wlin, r1(blin))
